# baseline (speedup 1.0000x reference)
import functools

import jax
import jax.numpy as jnp
from jax import lax
from jax.experimental import pallas as pl
from jax.experimental.pallas import tpu as pltpu
from jax.experimental.pallas import tpu_sc as plsc

N = 10001
E = 160000
NP = 10240
NBLK = NP // 256
KGE = 128
BERT = 128
HID = 256
OUT = 128
B = 4096

NC = 2
NS = 16
NW = NC * NS
ROWS_PER_TILE = NP // NS
ER = 1280
CW = 128
F32 = jnp.float32
I32 = jnp.int32

_MESH = plsc.VectorSubcoreMesh(core_axis_name="c", subcore_axis_name="s")


def _zero_fill(ref, nrows):
    def zrow(r, carry):
        for k in range(8):
            ref[r, pl.ds(k * 16, 16)] = jnp.zeros((16,), F32)
        return carry
    lax.fori_loop(0, nrows, zrow, 0)


def _fill_identity_idx(idxbuf, base):
    iota = lax.iota(I32, 16)
    for j in range(ROWS_PER_TILE // CW):
        for k in range(CW // 16):
            idxbuf[j, pl.ds(k * 16, 16)] = iota + (base + j * CW + k * 16)


def _zero_accum_streamed(zrows, idxbuf, accum):
    for j in range(ROWS_PER_TILE // CW):
        pltpu.sync_copy(zrows, accum.at[idxbuf.at[j]])


def _copyout_streamed(accum, idxbuf, rows, out_view, sem, s):
    for j in range(ROWS_PER_TILE // CW):
        pltpu.async_copy(accum.at[idxbuf.at[j]], rows, sem).wait()
        pltpu.sync_copy(rows,
                        out_view.at[pl.ds(s * ROWS_PER_TILE + j * CW, CW)])


def _edge_loop(p_hbm, srcbuf, dstbuf, rows, accum, sem, nrows):
    def chunk(j, carry):
        pltpu.async_copy(p_hbm.at[srcbuf.at[j]], rows, sem).wait()
        pltpu.sync_copy(rows, accum.at[dstbuf.at[j]], add=True)
        return carry

    lax.fori_loop(0, nrows, chunk, 0)


@functools.partial(
    pl.kernel,
    out_type=jax.ShapeDtypeStruct((NC, NP, 16), F32),
    mesh=_MESH,
    scratch_types=[
        pltpu.VMEM((ER // NW, CW), I32),
        pltpu.VMEM((CW, 16), F32),
        pltpu.VMEM((CW, 16), F32),
        pltpu.VMEM((ROWS_PER_TILE // CW, CW), I32),
        pltpu.VMEM_SHARED((NP, 16), F32),
        pltpu.SemaphoreType.DMA,
    ],
)
def _deg_kernel(dst_hbm, out_hbm, dstbuf, ones_v, zer_v, idxbuf, accum, ssem):
    c = lax.axis_index("c")
    s = lax.axis_index("s")
    w = s * NC + c

    def orow(r, carry):
        ones_v[r, :] = jnp.full((16,), 1.0, F32)
        zer_v[r, :] = jnp.zeros((16,), F32)
        return carry

    lax.fori_loop(0, CW, orow, 0)
    _fill_identity_idx(idxbuf, s * ROWS_PER_TILE)
    _zero_accum_streamed(zer_v, idxbuf, accum)
    pltpu.sync_copy(dst_hbm.at[pl.ds(w * (ER // NW), ER // NW)], dstbuf)
    plsc.subcore_barrier()

    def chunk(j, carry):
        pltpu.async_copy(ones_v, accum.at[dstbuf.at[j]], ssem, add=True)
        return carry

    lax.fori_loop(0, ER // NW, chunk, 0)

    def drain(j, carry):
        pltpu.make_async_copy(ones_v, accum.at[dstbuf.at[0]], ssem).wait()
        return carry

    lax.fori_loop(0, ER // NW, drain, 0)
    plsc.subcore_barrier()
    _copyout_streamed(accum, idxbuf, zer_v, out_hbm.at[c], ssem, s)


def _mm1_body(kge_ref, bert_ref, w1_ref, deg_ref, p1_ref, s1_ref):
    d = deg_ref[0, :, 0:1] + deg_ref[1, :, 0:1] + 1.0
    rs = lax.rsqrt(d)
    sn = 1.0 / d
    sup = jnp.dot(kge_ref[...], w1_ref[0:KGE, :],
                  preferred_element_type=F32)
    sup = sup + jnp.dot(bert_ref[...], w1_ref[KGE:KGE + BERT, :],
                        preferred_element_type=F32)
    p1_ref[...] = sup * rs
    s1_ref[...] = sup * sn


def _mm1(kge_p, bert_p, W1, deg2):
    return pl.pallas_call(
        _mm1_body,
        grid=(NBLK, 2),
        in_specs=[
            pl.BlockSpec((256, KGE), lambda i, f: (i, 0)),
            pl.BlockSpec((256, BERT), lambda i, f: (i, 0)),
            pl.BlockSpec((HID, 128), lambda i, f: (0, f)),
            pl.BlockSpec((NC, 256, 16), lambda i, f: (0, i, 0)),
        ],
        out_specs=[
            pl.BlockSpec((256, 128), lambda i, f: (f * NBLK + i, 0)),
            pl.BlockSpec((256, 128), lambda i, f: (i, f)),
        ],
        out_shape=[
            jax.ShapeDtypeStruct((2 * NP, 128), F32),
            jax.ShapeDtypeStruct((NP, HID), F32),
        ],
    )(kge_p, bert_p, W1, deg2)


@functools.partial(
    pl.kernel,
    out_type=jax.ShapeDtypeStruct((NC, NP, 128), F32),
    mesh=_MESH,
    scratch_types=[
        pltpu.VMEM((ER // NS, CW), I32),
        pltpu.VMEM((ER // NS, CW), I32),
        pltpu.VMEM((CW, 128), F32),
        pltpu.VMEM((ROWS_PER_TILE // CW, CW), I32),
        pltpu.VMEM_SHARED((NP, 128), F32),
        pltpu.SemaphoreType.DMA,
    ],
)
def _spmm1_kernel(p_hbm, src_hbm, dst_hbm, out_hbm,
                  srcbuf, dstbuf, rows, idxbuf, accum, sem):
    c = lax.axis_index("c")
    s = lax.axis_index("s")
    nrows = ER // NS

    _fill_identity_idx(idxbuf, s * ROWS_PER_TILE)
    _zero_fill(rows, CW)
    _zero_accum_streamed(rows, idxbuf, accum)
    pltpu.sync_copy(src_hbm.at[pl.ds(s * nrows, nrows)], srcbuf)
    pltpu.sync_copy(dst_hbm.at[pl.ds(s * nrows, nrows)], dstbuf)

    offv = jnp.broadcast_to(c * NP, (16,)).astype(I32)

    def arow(r, carry):
        for k in range(8):
            srcbuf[r, pl.ds(k * 16, 16)] = srcbuf[r, pl.ds(k * 16, 16)] + offv
        return carry

    lax.fori_loop(0, nrows, arow, 0)
    plsc.subcore_barrier()
    _edge_loop(p_hbm, srcbuf, dstbuf, rows, accum, sem, nrows)
    plsc.subcore_barrier()
    _copyout_streamed(accum, idxbuf, rows, out_hbm.at[c], sem, s)


def _mm2_body(t1_ref, s1_ref, deg_ref, b1_ref, w2_ref, p2_ref, s2_ref):
    d = deg_ref[0, :, 0:1] + deg_ref[1, :, 0:1] + 1.0
    rs = lax.rsqrt(d)
    sn = 1.0 / d
    x1a = jax.nn.relu(rs * t1_ref[0] + s1_ref[:, 0:128] + b1_ref[:, 0:128])
    x1b = jax.nn.relu(rs * t1_ref[1] + s1_ref[:, 128:256] + b1_ref[:, 128:256])
    sup = jnp.dot(x1a, w2_ref[0:128, :], preferred_element_type=F32)
    sup = sup + jnp.dot(x1b, w2_ref[128:256, :], preferred_element_type=F32)
    p2_ref[...] = sup * rs
    s2_ref[...] = sup * sn


def _mm2(tmp1, S1, deg2, b1r, W2):
    return pl.pallas_call(
        _mm2_body,
        grid=(NBLK,),
        in_specs=[
            pl.BlockSpec((NC, 256, 128), lambda i: (0, i, 0)),
            pl.BlockSpec((256, HID), lambda i: (i, 0)),
            pl.BlockSpec((NC, 256, 16), lambda i: (0, i, 0)),
            pl.BlockSpec((1, HID), lambda i: (0, 0)),
            pl.BlockSpec((HID, OUT), lambda i: (0, 0)),
        ],
        out_specs=[
            pl.BlockSpec((256, OUT), lambda i: (i, 0)),
            pl.BlockSpec((256, OUT), lambda i: (i, 0)),
        ],
        out_shape=[
            jax.ShapeDtypeStruct((NP, OUT), F32),
            jax.ShapeDtypeStruct((NP, OUT), F32),
        ],
    )(tmp1, S1, deg2, b1r, W2)


@functools.partial(
    pl.kernel,
    out_type=jax.ShapeDtypeStruct((NC, NP, OUT), F32),
    mesh=_MESH,
    scratch_types=[
        pltpu.VMEM((ER // NW, CW), I32),
        pltpu.VMEM((ER // NW, CW), I32),
        pltpu.VMEM((CW, OUT), F32),
        pltpu.VMEM((80, OUT), F32),
        pltpu.VMEM_SHARED((NP, OUT), F32),
        pltpu.SemaphoreType.DMA,
    ],
)
def _spmm2_kernel(p_hbm, src_hbm, dst_hbm, out_hbm,
                  srcbuf, dstbuf, rows, zer, accum, sem):
    c = lax.axis_index("c")
    s = lax.axis_index("s")
    w = s * NC + c
    nrows = ER // NW

    _zero_fill(zer, 80)
    for b in range(ROWS_PER_TILE // 80):
        pltpu.sync_copy(zer, accum.at[pl.ds(s * ROWS_PER_TILE + b * 80, 80)])
    pltpu.sync_copy(src_hbm.at[pl.ds(w * nrows, nrows)], srcbuf)
    pltpu.sync_copy(dst_hbm.at[pl.ds(w * nrows, nrows)], dstbuf)

    plsc.subcore_barrier()
    _edge_loop(p_hbm, srcbuf, dstbuf, rows, accum, sem, nrows)
    plsc.subcore_barrier()
    pltpu.sync_copy(
        accum.at[pl.ds(s * ROWS_PER_TILE, ROWS_PER_TILE)],
        out_hbm.at[c].at[pl.ds(s * ROWS_PER_TILE, ROWS_PER_TILE)],
    )


def _fin_body(t2_ref, s2_ref, deg_ref, b2_ref, x2_ref):
    d = deg_ref[0, :, 0:1] + deg_ref[1, :, 0:1] + 1.0
    rs = lax.rsqrt(d)
    x2_ref[...] = jax.nn.relu(rs * (t2_ref[0] + t2_ref[1]) + s2_ref[...]
                              + b2_ref[...])


def _fin(tmp2, S2, deg2, b2r):
    return pl.pallas_call(
        _fin_body,
        grid=(NBLK,),
        in_specs=[
            pl.BlockSpec((NC, 256, OUT), lambda i: (0, i, 0)),
            pl.BlockSpec((256, OUT), lambda i: (i, 0)),
            pl.BlockSpec((NC, 256, 16), lambda i: (0, i, 0)),
            pl.BlockSpec((1, OUT), lambda i: (0, 0)),
        ],
        out_specs=pl.BlockSpec((256, OUT), lambda i: (i, 0)),
        out_shape=jax.ShapeDtypeStruct((NP, OUT), F32),
    )(tmp2, S2, deg2, b2r)


@functools.partial(
    pl.kernel,
    out_type=jax.ShapeDtypeStruct((B, OUT), F32),
    mesh=_MESH,
    scratch_types=[
        pltpu.VMEM((B // NW,), I32),
        pltpu.VMEM((B // NW, OUT), F32),
        pltpu.SemaphoreType.DMA,
    ],
)
def _take_kernel(x2_hbm, idx_hbm, out_hbm, idxbuf, rows, sem):
    c = lax.axis_index("c")
    s = lax.axis_index("s")
    w = s * NC + c
    pltpu.sync_copy(idx_hbm.at[w], idxbuf)
    pltpu.async_copy(x2_hbm.at[idxbuf], rows, sem).wait()
    pltpu.sync_copy(rows, out_hbm.at[pl.ds(w * (B // NW), B // NW)])


def kernel(entities_idx, edge_index, ent_kge, ent_abs_bert, W1, b1, W2, b2):
    src = edge_index[0].reshape(ER, 125)
    dst = edge_index[1].reshape(ER, 125)
    srcr = jnp.pad(src, ((0, 0), (0, CW - 125)))
    dstr = jnp.pad(dst, ((0, 0), (0, CW - 125)), constant_values=N)
    kge_p = jnp.pad(ent_kge, ((0, NP - N), (0, 0)))
    bert_p = jnp.pad(ent_abs_bert, ((0, NP - N), (0, 0)))
    idx2d = entities_idx.reshape(NW, B // NW)

    deg2 = _deg_kernel(dstr)
    P1, S1 = _mm1(kge_p, bert_p, W1, deg2)
    tmp1 = _spmm1_kernel(P1, srcr, dstr)
    P2, S2 = _mm2(tmp1, S1, deg2, b1.reshape(1, HID), W2)
    tmp2 = jnp.pad(_spmm2_kernel(P2, srcr, dstr), ((0,0),(0,NP-2048),(0,0)))
    x2 = _fin(tmp2, S2, deg2, b2.reshape(1, OUT))
    return _take_kernel(x2, idx2d)


# speedup vs baseline: 1.0974x; 1.0974x over previous
import functools

import jax
import jax.numpy as jnp
from jax import lax
from jax.experimental import pallas as pl
from jax.experimental.pallas import tpu as pltpu
from jax.experimental.pallas import tpu_sc as plsc

N = 10001
E = 160000
NP = 10240
NBLK = NP // 256
KGE = 128
BERT = 128
HID = 256
OUT = 128
B = 4096

NC = 2
NS = 16
NW = NC * NS
ROWS_PER_TILE = NP // NS
ER = 1280
CW = 128
F32 = jnp.float32
I32 = jnp.int32

_MESH = plsc.VectorSubcoreMesh(core_axis_name="c", subcore_axis_name="s")


def _zero_fill(ref, nrows):
    def zrow(r, carry):
        for k in range(8):
            ref[r, pl.ds(k * 16, 16)] = jnp.zeros((16,), F32)
        return carry
    lax.fori_loop(0, nrows, zrow, 0)


def _fill_identity_idx(idxbuf, base):
    iota = lax.iota(I32, 16)
    for j in range(ROWS_PER_TILE // CW):
        for k in range(CW // 16):
            idxbuf[j, pl.ds(k * 16, 16)] = iota + (base + j * CW + k * 16)


def _zero_accum_streamed(zrows, idxbuf, accum):
    for j in range(ROWS_PER_TILE // CW):
        pltpu.sync_copy(zrows, accum.at[idxbuf.at[j]])


def _copyout_streamed(accum, idxbuf, rows, out_view, sem, s):
    for j in range(ROWS_PER_TILE // CW):
        pltpu.async_copy(accum.at[idxbuf.at[j]], rows, sem).wait()
        pltpu.sync_copy(rows,
                        out_view.at[pl.ds(s * ROWS_PER_TILE + j * CW, CW)])


def _edge_loop(p_hbm, srcbuf, dstbuf, rows, accum, sem, nrows):
    def chunk(j, carry):
        pltpu.async_copy(p_hbm.at[srcbuf.at[j]], rows, sem).wait()
        pltpu.sync_copy(rows, accum.at[dstbuf.at[j]], add=True)
        return carry

    lax.fori_loop(0, nrows, chunk, 0)


@functools.partial(
    pl.kernel,
    out_type=jax.ShapeDtypeStruct((NC, NP, 16), F32),
    mesh=_MESH,
    scratch_types=[
        pltpu.VMEM((ER // NW, CW), I32),
        pltpu.VMEM((CW, 16), F32),
        pltpu.VMEM((CW, 16), F32),
        pltpu.VMEM((ROWS_PER_TILE // CW, CW), I32),
        pltpu.VMEM_SHARED((NP, 16), F32),
        pltpu.SemaphoreType.DMA,
    ],
)
def _deg_kernel(dst_hbm, out_hbm, dstbuf, ones_v, zer_v, idxbuf, accum, ssem):
    c = lax.axis_index("c")
    s = lax.axis_index("s")
    w = s * NC + c

    def orow(r, carry):
        ones_v[r, :] = jnp.full((16,), 1.0, F32)
        zer_v[r, :] = jnp.zeros((16,), F32)
        return carry

    lax.fori_loop(0, CW, orow, 0)
    _fill_identity_idx(idxbuf, s * ROWS_PER_TILE)
    _zero_accum_streamed(zer_v, idxbuf, accum)
    pltpu.sync_copy(dst_hbm.at[pl.ds(w * (ER // NW), ER // NW)], dstbuf)
    plsc.subcore_barrier()

    def chunk(j, carry):
        pltpu.async_copy(ones_v, accum.at[dstbuf.at[j]], ssem, add=True)
        return carry

    lax.fori_loop(0, ER // NW, chunk, 0)

    def drain(j, carry):
        pltpu.make_async_copy(ones_v, accum.at[dstbuf.at[0]], ssem).wait()
        return carry

    lax.fori_loop(0, ER // NW, drain, 0)
    plsc.subcore_barrier()
    _copyout_streamed(accum, idxbuf, zer_v, out_hbm.at[c], ssem, s)


def _mm1_body(kge_ref, bert_ref, w1_ref, deg_ref, p1_ref, s1_ref):
    d = deg_ref[0, :, 0:1] + deg_ref[1, :, 0:1] + 1.0
    rs = lax.rsqrt(d)
    sn = 1.0 / d
    sup = jnp.dot(kge_ref[...], w1_ref[0:KGE, :],
                  preferred_element_type=F32)
    sup = sup + jnp.dot(bert_ref[...], w1_ref[KGE:KGE + BERT, :],
                        preferred_element_type=F32)
    p = sup * rs
    p1_ref[0] = p[:, 0:128]
    p1_ref[1] = p[:, 128:256]
    s1_ref[...] = sup * sn


def _mm1(kge_p, bert_p, W1, deg2):
    return pl.pallas_call(
        _mm1_body,
        grid=(NBLK,),
        in_specs=[
            pl.BlockSpec((256, KGE), lambda i: (i, 0)),
            pl.BlockSpec((256, BERT), lambda i: (i, 0)),
            pl.BlockSpec((HID, HID), lambda i: (0, 0)),
            pl.BlockSpec((NC, 256, 16), lambda i: (0, i, 0)),
        ],
        out_specs=[
            pl.BlockSpec((2, 256, 128), lambda i: (0, i, 0)),
            pl.BlockSpec((256, HID), lambda i: (i, 0)),
        ],
        out_shape=[
            jax.ShapeDtypeStruct((2, NP, 128), F32),
            jax.ShapeDtypeStruct((NP, HID), F32),
        ],
    )(kge_p, bert_p, W1, deg2)


@functools.partial(
    pl.kernel,
    out_type=jax.ShapeDtypeStruct((NC, NP, 128), F32),
    mesh=_MESH,
    scratch_types=[
        pltpu.VMEM((ER // NS, CW), I32),
        pltpu.VMEM((ER // NS, CW), I32),
        pltpu.VMEM((CW, 128), F32),
        pltpu.VMEM((ROWS_PER_TILE // CW, CW), I32),
        pltpu.VMEM_SHARED((NP, 128), F32),
        pltpu.SemaphoreType.DMA,
    ],
)
def _spmm1_kernel(p_hbm, src_hbm, dst_hbm, out_hbm,
                  srcbuf, dstbuf, rows, idxbuf, accum, sem):
    c = lax.axis_index("c")
    s = lax.axis_index("s")
    nrows = ER // NS

    _fill_identity_idx(idxbuf, s * ROWS_PER_TILE)
    _zero_fill(rows, CW)
    _zero_accum_streamed(rows, idxbuf, accum)
    pltpu.sync_copy(src_hbm.at[pl.ds(s * nrows, nrows)], srcbuf)
    pltpu.sync_copy(dst_hbm.at[pl.ds(s * nrows, nrows)], dstbuf)

    offv = jnp.broadcast_to(c * NP, (16,)).astype(I32)

    def arow(r, carry):
        for k in range(8):
            srcbuf[r, pl.ds(k * 16, 16)] = srcbuf[r, pl.ds(k * 16, 16)] + offv
        return carry

    lax.fori_loop(0, nrows, arow, 0)
    plsc.subcore_barrier()
    _edge_loop(p_hbm, srcbuf, dstbuf, rows, accum, sem, nrows)
    plsc.subcore_barrier()
    _copyout_streamed(accum, idxbuf, rows, out_hbm.at[c], sem, s)


def _mm2_body(t1_ref, s1_ref, deg_ref, b1_ref, w2_ref, p2_ref, s2_ref,
              rs_ref):
    d = deg_ref[0, :, 0:1] + deg_ref[1, :, 0:1] + 1.0
    rs = lax.rsqrt(d)
    sn = 1.0 / d
    x1a = jax.nn.relu(rs * t1_ref[0] + s1_ref[:, 0:128] + b1_ref[:, 0:128])
    x1b = jax.nn.relu(rs * t1_ref[1] + s1_ref[:, 128:256] + b1_ref[:, 128:256])
    sup = jnp.dot(x1a, w2_ref[0:128, :], preferred_element_type=F32)
    sup = sup + jnp.dot(x1b, w2_ref[128:256, :], preferred_element_type=F32)
    p2_ref[...] = sup * rs
    s2_ref[...] = sup * sn
    rs_ref[...] = jnp.broadcast_to(rs, (256, 128))


def _mm2(tmp1, S1, deg2, b1r, W2):
    return pl.pallas_call(
        _mm2_body,
        grid=(NBLK,),
        in_specs=[
            pl.BlockSpec((NC, 256, 128), lambda i: (0, i, 0)),
            pl.BlockSpec((256, HID), lambda i: (i, 0)),
            pl.BlockSpec((NC, 256, 16), lambda i: (0, i, 0)),
            pl.BlockSpec((1, HID), lambda i: (0, 0)),
            pl.BlockSpec((HID, OUT), lambda i: (0, 0)),
        ],
        out_specs=[
            pl.BlockSpec((256, OUT), lambda i: (i, 0)),
            pl.BlockSpec((256, OUT), lambda i: (i, 0)),
            pl.BlockSpec((256, 128), lambda i: (i, 0)),
        ],
        out_shape=[
            jax.ShapeDtypeStruct((NP, OUT), F32),
            jax.ShapeDtypeStruct((NP, OUT), F32),
            jax.ShapeDtypeStruct((NP, 128), F32),
        ],
    )(tmp1, S1, deg2, b1r, W2)


@functools.partial(
    pl.kernel,
    out_type=jax.ShapeDtypeStruct((NC, NP, OUT), F32),
    mesh=_MESH,
    scratch_types=[
        pltpu.VMEM((ER // NW, CW), I32),
        pltpu.VMEM((ER // NW, CW), I32),
        pltpu.VMEM((CW, OUT), F32),
        pltpu.VMEM((80, OUT), F32),
        pltpu.VMEM_SHARED((NP, OUT), F32),
        pltpu.SemaphoreType.DMA,
    ],
)
def _spmm2_kernel(p_hbm, src_hbm, dst_hbm, out_hbm,
                  srcbuf, dstbuf, rows, zer, accum, sem):
    c = lax.axis_index("c")
    s = lax.axis_index("s")
    w = s * NC + c
    nrows = ER // NW

    _zero_fill(zer, 80)
    for b in range(ROWS_PER_TILE // 80):
        pltpu.sync_copy(zer, accum.at[pl.ds(s * ROWS_PER_TILE + b * 80, 80)])
    pltpu.sync_copy(src_hbm.at[pl.ds(w * nrows, nrows)], srcbuf)
    pltpu.sync_copy(dst_hbm.at[pl.ds(w * nrows, nrows)], dstbuf)

    plsc.subcore_barrier()
    _edge_loop(p_hbm, srcbuf, dstbuf, rows, accum, sem, nrows)
    plsc.subcore_barrier()
    pltpu.sync_copy(
        accum.at[pl.ds(s * ROWS_PER_TILE, ROWS_PER_TILE)],
        out_hbm.at[c].at[pl.ds(s * ROWS_PER_TILE, ROWS_PER_TILE)],
    )


def _fin_body(t2_ref, s2_ref, deg_ref, b2_ref, x2_ref):
    d = deg_ref[0, :, 0:1] + deg_ref[1, :, 0:1] + 1.0
    rs = lax.rsqrt(d)
    x2_ref[...] = jax.nn.relu(rs * (t2_ref[0] + t2_ref[1]) + s2_ref[...]
                              + b2_ref[...])


def _fin(tmp2, S2, deg2, b2r):
    return pl.pallas_call(
        _fin_body,
        grid=(NBLK,),
        in_specs=[
            pl.BlockSpec((NC, 256, OUT), lambda i: (0, i, 0)),
            pl.BlockSpec((256, OUT), lambda i: (i, 0)),
            pl.BlockSpec((NC, 256, 16), lambda i: (0, i, 0)),
            pl.BlockSpec((1, OUT), lambda i: (0, 0)),
        ],
        out_specs=pl.BlockSpec((256, OUT), lambda i: (i, 0)),
        out_shape=jax.ShapeDtypeStruct((NP, OUT), F32),
    )(tmp2, S2, deg2, b2r)


@functools.partial(
    pl.kernel,
    out_type=jax.ShapeDtypeStruct((B, OUT), F32),
    mesh=_MESH,
    scratch_types=[
        pltpu.VMEM((B // NW,), I32),
        pltpu.VMEM((B // NW,), I32),
        pltpu.VMEM((B // NW, OUT), F32),
        pltpu.VMEM((B // NW, OUT), F32),
        pltpu.VMEM((B // NW, OUT), F32),
        pltpu.VMEM((B // NW, 128), F32),
        pltpu.VMEM((OUT,), F32),
        pltpu.SemaphoreType.DMA,
    ],
)
def _take_kernel(t2_hbm, s2_hbm, rs_hbm, b2_hbm, idx_hbm, out_hbm,
                 idxb, idxb2, t0, t1, s2b, rsb, b2b, sem):
    c = lax.axis_index("c")
    s = lax.axis_index("s")
    w = s * NC + c
    nr = B // NW
    pltpu.sync_copy(idx_hbm.at[w], idxb)
    pltpu.sync_copy(b2_hbm, b2b)
    for k in range(nr // 16):
        idxb2[pl.ds(k * 16, 16)] = idxb[pl.ds(k * 16, 16)] + NP
    pltpu.async_copy(t2_hbm.at[idxb], t0, sem).wait()
    pltpu.async_copy(t2_hbm.at[idxb2], t1, sem).wait()
    pltpu.async_copy(s2_hbm.at[idxb], s2b, sem).wait()
    pltpu.async_copy(rs_hbm.at[idxb], rsb, sem).wait()

    def row(i, carry):
        rsv = rsb[i, pl.ds(0, 16)]
        for k in range(OUT // 16):
            sl = pl.ds(k * 16, 16)
            v = rsv * (t0[i, sl] + t1[i, sl]) + s2b[i, sl] + b2b[sl]
            t0[i, sl] = jnp.maximum(v, 0.0)
        return carry

    lax.fori_loop(0, nr, row, 0)
    pltpu.sync_copy(t0, out_hbm.at[pl.ds(w * nr, nr)])


def kernel(entities_idx, edge_index, ent_kge, ent_abs_bert, W1, b1, W2, b2):
    src = edge_index[0].reshape(ER, 125)
    dst = edge_index[1].reshape(ER, 125)
    srcr = jnp.pad(src, ((0, 0), (0, CW - 125)))
    dstr = jnp.pad(dst, ((0, 0), (0, CW - 125)), constant_values=N)
    kge_p = jnp.pad(ent_kge, ((0, NP - N), (0, 0)))
    bert_p = jnp.pad(ent_abs_bert, ((0, NP - N), (0, 0)))
    idx2d = entities_idx.reshape(NW, B // NW)

    deg2 = _deg_kernel(dstr)
    P1, S1 = _mm1(kge_p, bert_p, W1, deg2)
    tmp1 = _spmm1_kernel(P1.reshape(2 * NP, 128), srcr, dstr)
    P2, S2, rs2 = _mm2(tmp1, S1, deg2, b1.reshape(1, HID), W2)
    tmp2 = _spmm2_kernel(P2, srcr, dstr)
    return _take_kernel(tmp2.reshape(2 * NP, OUT), S2, rs2, b2, idx2d)
